# baseline (speedup 1.0000x reference)
import functools

import jax
import jax.numpy as jnp
from jax import lax
from jax.experimental import pallas as pl
from jax.experimental.pallas import tpu as pltpu
from jax.experimental.pallas import tpu_sc as plsc

USER = 3000
ITEM = 7000
N = USER + ITEM
D = 128
L = 2
B = 2
E = 320000
HOPS = 2

NC = 2
NS = 16
EPT = E // NS
C = 64
EPTP = 20480
NCHUNK = EPTP // C
SEG = 16
NSEG = NCHUNK // SEG
GRP = 4
NPAD = 10240
RPT = NPAD // NS


def _spmm2_body(xall, pk, vals, zeros, cur1, cur2,
                idx_v, val_v, rows0, rows1, rows2, rows3,
                acc_s, g0, g1, g2, g3, s0, s1, s2, s3):
    l = lax.axis_index("c")
    w = lax.axis_index("s")
    rows = (rows0, rows1, rows2, rows3)
    gs = (g0, g1, g2, g3)
    ss = (s0, s1, s2, s3)
    for b in range(B):
        for hop in range(HOPS):
            if hop == 0:
                table = xall.at[pl.ds(l * N, N)]
            else:
                table = cur1.at[pl.ds((b * L + l) * NPAD, NPAD)]
            out = cur1 if hop == 0 else cur2
            gbase = ((b * NS) + w) * NSEG
            pltpu.sync_copy(zeros, acc_s.at[pl.ds(w * RPT, RPT)])
            plsc.subcore_barrier()

            def seg(s, _, table=table):
                @pl.when(s > 0)
                def _():
                    for u in range(GRP):
                        pltpu.make_async_copy(
                            rows[u], acc_s.at[idx_v.at[0, 0]],
                            ss[u]).wait()

                pltpu.sync_copy(pk.at[gbase + s], idx_v)
                pltpu.sync_copy(vals.at[gbase + s], val_v)

                def group(t, _, table=table):
                    for u in range(GRP):
                        jl = t * GRP + u

                        @pl.when(t > 0)
                        def _(u=u):
                            pltpu.make_async_copy(
                                rows[u], acc_s.at[idx_v.at[0, 0]],
                                ss[u]).wait()

                        pltpu.async_copy(table.at[idx_v.at[jl, 1]],
                                         rows[u], gs[u])
                    for u in range(GRP):
                        jl = t * GRP + u
                        pltpu.make_async_copy(
                            table.at[idx_v.at[0, 1]], rows[u],
                            gs[u]).wait()

                        def scale16(g, _, u=u, jl=jl):
                            vv = val_v[jl, pl.ds(g * 16, 16)]
                            for i in range(16):
                                cc = g * 16 + i
                                v = vv[i]
                                for k in range(D // 16):
                                    sl = pl.ds(k * 16, 16)
                                    rows[u][cc, sl] = rows[u][cc, sl] * v
                            return ()

                        lax.fori_loop(0, C // 16, scale16, ())
                        pltpu.async_copy(rows[u],
                                         acc_s.at[idx_v.at[jl, 0]],
                                         ss[u], add=True)
                    return ()

                lax.fori_loop(0, SEG // GRP, group, ())
                return ()

            lax.fori_loop(0, NSEG, seg, ())
            for u in range(GRP):
                pltpu.make_async_copy(rows[u], acc_s.at[idx_v.at[0, 0]],
                                      ss[u]).wait()
            plsc.subcore_barrier()
            base = (b * L + l) * NPAD
            pltpu.sync_copy(acc_s.at[pl.ds(w * RPT, RPT)],
                            out.at[pl.ds(base + w * RPT, RPT)])
            plsc.subcore_barrier()


@functools.cache
def _get_spmm2():
    mesh = plsc.VectorSubcoreMesh(
        core_axis_name="c", subcore_axis_name="s",
        num_cores=NC, num_subcores=NS)
    return pl.kernel(
        _spmm2_body,
        out_type=[
            jax.ShapeDtypeStruct((B * L * NPAD, D), jnp.float32),
            jax.ShapeDtypeStruct((B * L * NPAD, D), jnp.float32),
        ],
        mesh=mesh,
        scratch_types=[
            pltpu.VMEM((SEG, 2, C), jnp.int32),
            pltpu.VMEM((SEG, C), jnp.float32),
            pltpu.VMEM((C, D), jnp.float32),
            pltpu.VMEM((C, D), jnp.float32),
            pltpu.VMEM((C, D), jnp.float32),
            pltpu.VMEM((C, D), jnp.float32),
            pltpu.VMEM_SHARED((NPAD, D), jnp.float32),
            pltpu.SemaphoreType.DMA,
            pltpu.SemaphoreType.DMA,
            pltpu.SemaphoreType.DMA,
            pltpu.SemaphoreType.DMA,
            pltpu.SemaphoreType.DMA,
            pltpu.SemaphoreType.DMA,
            pltpu.SemaphoreType.DMA,
            pltpu.SemaphoreType.DMA,
        ],
    )


BLK = 1000
NBLK = N // BLK
UBLK = USER // BLK


def _stats_body(x0_ref, c1_ref, c2_ref, acc_ref, su_ref, si_ref):
    r = pl.program_id(2)
    a = x0_ref[0] + c1_ref[0, 0] + c2_ref[0, 0]
    acc_ref[0, 0] = a
    p = lax.dot_general(a, a, (((0,), (0,)), ((), ())),
                        precision=lax.Precision.HIGHEST)

    @pl.when(r == 0)
    def _():
        su_ref[...] = jnp.zeros_like(su_ref)
        si_ref[...] = jnp.zeros_like(si_ref)

    is_user = (r < UBLK).astype(jnp.float32)
    su_ref[0, 0] += is_user * p
    si_ref[0, 0] += (1.0 - is_user) * p


_stats = pl.pallas_call(
    _stats_body,
    grid=(L, B, NBLK),
    in_specs=[
        pl.BlockSpec((1, BLK, D), lambda l, b, r: (l, r, 0)),
        pl.BlockSpec((1, 1, BLK, D), lambda l, b, r: (b, l, r, 0)),
        pl.BlockSpec((1, 1, BLK, D), lambda l, b, r: (b, l, r, 0)),
    ],
    out_specs=[
        pl.BlockSpec((1, 1, BLK, D), lambda l, b, r: (l, b, r, 0)),
        pl.BlockSpec((1, 1, D, D), lambda l, b, r: (l, b, 0, 0)),
        pl.BlockSpec((1, 1, D, D), lambda l, b, r: (l, b, 0, 0)),
    ],
    out_shape=[
        jax.ShapeDtypeStruct((L, B, N, D), jnp.float32),
        jax.ShapeDtypeStruct((L, B, D, D), jnp.float32),
        jax.ShapeDtypeStruct((L, B, D, D), jnp.float32),
    ],
)


def _mid_body(uh_ref, ih_ref, su_ref, si_ref, mu_ref, mi_ref):
    for l in range(L):
        gu = lax.dot_general(uh_ref[l], uh_ref[l], (((0,), (0,)), ((), ())),
                             precision=lax.Precision.HIGHEST)
        gi = lax.dot_general(ih_ref[l], ih_ref[l], (((0,), (0,)), ((), ())),
                             precision=lax.Precision.HIGHEST)
        for b in range(B):
            mu_ref[l, b] = jnp.dot(gu, su_ref[l, b],
                                   precision=lax.Precision.HIGHEST)
            mi_ref[l, b] = jnp.dot(gi, si_ref[l, b],
                                   precision=lax.Precision.HIGHEST)


_mid = pl.pallas_call(
    _mid_body,
    out_shape=[
        jax.ShapeDtypeStruct((L, B, D, D), jnp.float32),
        jax.ShapeDtypeStruct((L, B, D, D), jnp.float32),
    ],
)


def _emit_body(acc_ref, m_ref, w_ref, cw_ref, emb_ref, embs_ref):
    emb = jnp.zeros((BLK, D), jnp.float32)
    embs = [jnp.zeros((BLK, D), jnp.float32) for _ in range(B)]
    for l in range(L):
        ts = []
        for b in range(B):
            t_lb = jnp.dot(acc_ref[l, b], m_ref[l, b],
                           precision=lax.Precision.HIGHEST)
            ts.append(t_lb)
            embs[b] = embs[b] + jnp.dot(
                jax.nn.relu(jnp.dot(t_lb, w_ref[l],
                                    precision=lax.Precision.HIGHEST)),
                cw_ref[l], precision=lax.Precision.HIGHEST)
        tm = (ts[0] + ts[1]) * (1.0 / B)
        emb = emb + jnp.dot(
            jax.nn.relu(jnp.dot(tm, w_ref[l],
                                precision=lax.Precision.HIGHEST)),
            cw_ref[l], precision=lax.Precision.HIGHEST)
    emb_ref[...] = emb
    for b in range(B):
        embs_ref[b] = embs[b]


def _make_emit(rows, row_off_blocks):
    nblk = rows // BLK
    return pl.pallas_call(
        _emit_body,
        grid=(nblk,),
        in_specs=[
            pl.BlockSpec((L, B, BLK, D), lambda r: (0, 0, r + row_off_blocks, 0)),
            pl.BlockSpec((L, B, D, D), lambda r: (0, 0, 0, 0)),
            pl.BlockSpec((L, D, D), lambda r: (0, 0, 0)),
            pl.BlockSpec((L, D, D), lambda r: (0, 0, 0)),
        ],
        out_specs=[
            pl.BlockSpec((BLK, D), lambda r: (r, 0)),
            pl.BlockSpec((B, BLK, D), lambda r: (0, r, 0)),
        ],
        out_shape=[
            jax.ShapeDtypeStruct((rows, D), jnp.float32),
            jax.ShapeDtypeStruct((B, rows, D), jnp.float32),
        ],
    )


_emit_user = _make_emit(USER, 0)
_emit_item = _make_emit(ITEM, UBLK)


def kernel(uEmbeds, iEmbeds, uHyper, iHyper, u_w, i_w,
           u_concat_w, i_concat_w, edge_vals, edge_index):
    x0 = jnp.concatenate([uEmbeds, iEmbeds], axis=1)
    xall = x0.reshape(L * N, D)

    pad = EPTP - EPT
    dst = edge_index[:, 0, :].reshape(B, NS, EPT)
    src = edge_index[:, 1, :].reshape(B, NS, EPT)
    zpad = jnp.zeros((B, NS, pad), jnp.int32)
    dst = jnp.concatenate([dst, zpad], axis=2)
    src = jnp.concatenate([src, zpad], axis=2)
    pk = jnp.stack([dst, src], axis=2)
    pk = pk.reshape(B, NS, 2, NSEG, SEG, C).transpose(0, 1, 3, 4, 2, 5)
    pk = pk.reshape(B * NS * NSEG, SEG, 2, C)
    valp = jnp.concatenate(
        [edge_vals.reshape(B, NS, EPT),
         jnp.zeros((B, NS, pad), jnp.float32)], axis=2)
    vals = valp.reshape(B * NS * NSEG, SEG, C)
    zeros = jnp.zeros((RPT, D), jnp.float32)

    cur1, cur2 = _get_spmm2()(xall, pk, vals, zeros)
    c1 = cur1.reshape(B, L, NPAD, D)[:, :, :N]
    c2 = cur2.reshape(B, L, NPAD, D)[:, :, :N]

    acc, su, si = _stats(x0, c1, c2)
    mu, mi = _mid(uHyper, iHyper, su, si)

    ucw = u_concat_w.reshape(L, D, D)
    icw = i_concat_w.reshape(L, D, D)
    ue, ues = _emit_user(acc, mu, u_w, ucw)
    ie, ies = _emit_item(acc, mi, i_w, icw)
    return ue, ie, ues, ies


# speedup vs baseline: 1.4437x; 1.4437x over previous
import functools

import jax
import jax.numpy as jnp
from jax import lax
from jax.experimental import pallas as pl
from jax.experimental.pallas import tpu as pltpu
from jax.experimental.pallas import tpu_sc as plsc

USER = 3000
K = 128
ITEM = 7000
N = USER + ITEM
D = 128
L = 2
B = 2
E = 320000
HOPS = 2

NC = 2
NS = 16
EPT = E // NS
C = 80
EPTP = 20160
NCHUNK = EPTP // C
SEG = 21
NSEG = NCHUNK // SEG
GRP = 3
NPAD = 10240
RPT = NPAD // NS


def _spmm2_body(xall, pk, vals, zeros, cur1, cur2,
                idx_v, val_v, rows0, rows1, rows2,
                acc_s, g0, g1, g2, s0, s1, s2):
    l = lax.axis_index("c")
    w = lax.axis_index("s")
    rows = (rows0, rows1, rows2)
    gs = (g0, g1, g2)
    ss = (s0, s1, s2)
    for b in range(B):
        for hop in range(HOPS):
            if hop == 0:
                table = xall.at[pl.ds(l * N, N)]
            else:
                table = cur1.at[pl.ds((b * L + l) * NPAD, NPAD)]
            out = cur1 if hop == 0 else cur2
            gbase = ((b * NS) + w) * NSEG
            pltpu.sync_copy(zeros, acc_s.at[pl.ds(w * RPT, RPT)])
            plsc.subcore_barrier()

            def seg(s, _, table=table):
                @pl.when(s > 0)
                def _():
                    for u in range(GRP):
                        pltpu.make_async_copy(
                            rows[u], acc_s.at[idx_v.at[0, 0]],
                            ss[u]).wait()

                pltpu.sync_copy(pk.at[gbase + s], idx_v)
                pltpu.sync_copy(vals.at[gbase + s], val_v)

                def group(t, _, table=table):
                    for u in range(GRP):
                        jl = t * GRP + u

                        @pl.when(t > 0)
                        def _(u=u):
                            pltpu.make_async_copy(
                                rows[u], acc_s.at[idx_v.at[0, 0]],
                                ss[u]).wait()

                        pltpu.async_copy(table.at[idx_v.at[jl, 1]],
                                         rows[u], gs[u])
                    for u in range(GRP):
                        jl = t * GRP + u
                        pltpu.make_async_copy(
                            table.at[idx_v.at[0, 1]], rows[u],
                            gs[u]).wait()

                        def scale16(g, _, u=u, jl=jl):
                            vv = val_v[jl, pl.ds(g * 16, 16)]
                            for i in range(16):
                                cc = g * 16 + i
                                v = vv[i]
                                for k in range(D // 16):
                                    sl = pl.ds(k * 16, 16)
                                    rows[u][cc, sl] = rows[u][cc, sl] * v
                            return ()

                        lax.fori_loop(0, C // 16, scale16, ())
                        pltpu.async_copy(rows[u],
                                         acc_s.at[idx_v.at[jl, 0]],
                                         ss[u], add=True)
                    return ()

                lax.fori_loop(0, SEG // GRP, group, ())
                return ()

            lax.fori_loop(0, NSEG, seg, ())
            for u in range(GRP):
                pltpu.make_async_copy(rows[u], acc_s.at[idx_v.at[0, 0]],
                                      ss[u]).wait()
            plsc.subcore_barrier()
            base = (b * L + l) * NPAD
            pltpu.sync_copy(acc_s.at[pl.ds(w * RPT, RPT)],
                            out.at[pl.ds(base + w * RPT, RPT)])
            plsc.subcore_barrier()


@functools.cache
def _get_spmm2():
    mesh = plsc.VectorSubcoreMesh(
        core_axis_name="c", subcore_axis_name="s",
        num_cores=NC, num_subcores=NS)
    return pl.kernel(
        _spmm2_body,
        out_type=[
            jax.ShapeDtypeStruct((B * L * NPAD, D), jnp.float32),
            jax.ShapeDtypeStruct((B * L * NPAD, D), jnp.float32),
        ],
        mesh=mesh,
        scratch_types=[
            pltpu.VMEM((SEG, 2, C), jnp.int32),
            pltpu.VMEM((SEG, C), jnp.float32),
            pltpu.VMEM((C, D), jnp.float32),
            pltpu.VMEM((C, D), jnp.float32),
            pltpu.VMEM((C, D), jnp.float32),
            pltpu.VMEM_SHARED((NPAD, D), jnp.float32),
            pltpu.SemaphoreType.DMA,
            pltpu.SemaphoreType.DMA,
            pltpu.SemaphoreType.DMA,
            pltpu.SemaphoreType.DMA,
            pltpu.SemaphoreType.DMA,
            pltpu.SemaphoreType.DMA,
        ],
    )


BLK = 1000
NBLK = N // BLK
UBLK = USER // BLK


def _stats_body(x0_ref, c1_ref, c2_ref, acc_ref, su_ref, si_ref):
    r = pl.program_id(2)
    a = x0_ref[0] + c1_ref[0, 0] + c2_ref[0, 0]
    acc_ref[0, 0] = a
    p = lax.dot_general(a, a, (((0,), (0,)), ((), ())),
                        precision=lax.Precision.HIGHEST)

    @pl.when(r == 0)
    def _():
        su_ref[...] = jnp.zeros_like(su_ref)
        si_ref[...] = jnp.zeros_like(si_ref)

    is_user = (r < UBLK).astype(jnp.float32)
    su_ref[0, 0] += is_user * p
    si_ref[0, 0] += (1.0 - is_user) * p


_stats = pl.pallas_call(
    _stats_body,
    grid=(L, B, NBLK),
    in_specs=[
        pl.BlockSpec((1, BLK, D), lambda l, b, r: (l, r, 0)),
        pl.BlockSpec((1, 1, BLK, D), lambda l, b, r: (b, l, r, 0)),
        pl.BlockSpec((1, 1, BLK, D), lambda l, b, r: (b, l, r, 0)),
    ],
    out_specs=[
        pl.BlockSpec((1, 1, BLK, D), lambda l, b, r: (l, b, r, 0)),
        pl.BlockSpec((1, 1, D, D), lambda l, b, r: (l, b, 0, 0)),
        pl.BlockSpec((1, 1, D, D), lambda l, b, r: (l, b, 0, 0)),
    ],
    out_shape=[
        jax.ShapeDtypeStruct((L, B, N, D), jnp.float32),
        jax.ShapeDtypeStruct((L, B, D, D), jnp.float32),
        jax.ShapeDtypeStruct((L, B, D, D), jnp.float32),
    ],
)


def _emit_body(acc_ref, hy_ref, s_ref, w_ref, cw_ref, emb_ref, embs_ref):
    emb = jnp.zeros((BLK, D), jnp.float32)
    embs = [jnp.zeros((BLK, D), jnp.float32) for _ in range(B)]
    for l in range(L):
        g = lax.dot_general(hy_ref[l], hy_ref[l], (((0,), (0,)), ((), ())),
                            precision=lax.Precision.HIGHEST)
        ts = []
        for b in range(B):
            m_lb = jnp.dot(g, s_ref[l, b], precision=lax.Precision.HIGHEST)
            t_lb = jnp.dot(acc_ref[l, b], m_lb,
                           precision=lax.Precision.HIGHEST)
            ts.append(t_lb)
            embs[b] = embs[b] + jnp.dot(
                jax.nn.relu(jnp.dot(t_lb, w_ref[l],
                                    precision=lax.Precision.HIGHEST)),
                cw_ref[l], precision=lax.Precision.HIGHEST)
        tm = (ts[0] + ts[1]) * (1.0 / B)
        emb = emb + jnp.dot(
            jax.nn.relu(jnp.dot(tm, w_ref[l],
                                precision=lax.Precision.HIGHEST)),
            cw_ref[l], precision=lax.Precision.HIGHEST)
    emb_ref[...] = emb
    for b in range(B):
        embs_ref[b] = embs[b]


def _make_emit(rows, row_off_blocks):
    nblk = rows // BLK
    return pl.pallas_call(
        _emit_body,
        grid=(nblk,),
        in_specs=[
            pl.BlockSpec((L, B, BLK, D), lambda r: (0, 0, r + row_off_blocks, 0)),
            pl.BlockSpec((L, K, D), lambda r: (0, 0, 0)),
            pl.BlockSpec((L, B, D, D), lambda r: (0, 0, 0, 0)),
            pl.BlockSpec((L, D, D), lambda r: (0, 0, 0)),
            pl.BlockSpec((L, D, D), lambda r: (0, 0, 0)),
        ],
        out_specs=[
            pl.BlockSpec((BLK, D), lambda r: (r, 0)),
            pl.BlockSpec((B, BLK, D), lambda r: (0, r, 0)),
        ],
        out_shape=[
            jax.ShapeDtypeStruct((rows, D), jnp.float32),
            jax.ShapeDtypeStruct((B, rows, D), jnp.float32),
        ],
    )


_emit_user = _make_emit(USER, 0)
_emit_item = _make_emit(ITEM, UBLK)


def kernel(uEmbeds, iEmbeds, uHyper, iHyper, u_w, i_w,
           u_concat_w, i_concat_w, edge_vals, edge_index):
    x0 = jnp.concatenate([uEmbeds, iEmbeds], axis=1)
    xall = x0.reshape(L * N, D)

    pad = EPTP - EPT
    dst = edge_index[:, 0, :].reshape(B, NS, EPT)
    src = edge_index[:, 1, :].reshape(B, NS, EPT)
    zpad = jnp.zeros((B, NS, pad), jnp.int32)
    dst = jnp.concatenate([dst, zpad], axis=2)
    src = jnp.concatenate([src, zpad], axis=2)
    pk = jnp.stack([dst, src], axis=2)
    pk = pk.reshape(B, NS, 2, NSEG, SEG, C).transpose(0, 1, 3, 4, 2, 5)
    pk = pk.reshape(B * NS * NSEG, SEG, 2, C)
    valp = jnp.concatenate(
        [edge_vals.reshape(B, NS, EPT),
         jnp.zeros((B, NS, pad), jnp.float32)], axis=2)
    vals = valp.reshape(B * NS * NSEG, SEG, C)
    zeros = jnp.zeros((RPT, D), jnp.float32)

    cur1, cur2 = _get_spmm2()(xall, pk, vals, zeros)
    c1 = cur1.reshape(B, L, NPAD, D)
    c2 = cur2.reshape(B, L, NPAD, D)

    acc, su, si = _stats(x0, c1, c2)

    ucw = u_concat_w.reshape(L, D, D)
    icw = i_concat_w.reshape(L, D, D)
    ue, ues = _emit_user(acc, uHyper, su, u_w, ucw)
    ie, ies = _emit_item(acc, iHyper, si, i_w, icw)
    return ue, ie, ues, ies


# speedup vs baseline: 1.4802x; 1.0253x over previous
import functools

import jax
import jax.numpy as jnp
from jax import lax
from jax.experimental import pallas as pl
from jax.experimental.pallas import tpu as pltpu
from jax.experimental.pallas import tpu_sc as plsc

USER = 3000
K = 128
ITEM = 7000
N = USER + ITEM
D = 128
L = 2
B = 2
E = 320000
HOPS = 2

NC = 2
NS = 16
EPT = E // NS
C = 80
EPTP = 20160
NCHUNK = EPTP // C
SEG = 42
NSEG = NCHUNK // SEG
GRP = 3
NPAD = 10240
RPT = NPAD // NS


def _spmm2_body(xall, pk, vals, zeros, cur1, cur2,
                idx_v, val_v, rows0, rows1, rows2,
                acc_s, g0, g1, g2, s0, s1, s2):
    l = lax.axis_index("c")
    w = lax.axis_index("s")
    rows = (rows0, rows1, rows2)
    gs = (g0, g1, g2)
    ss = (s0, s1, s2)
    for b in range(B):
        for hop in range(HOPS):
            if hop == 0:
                table = xall.at[pl.ds(l * N, N)]
            else:
                table = cur1.at[pl.ds((b * L + l) * NPAD, NPAD)]
            out = cur1 if hop == 0 else cur2
            gbase = ((b * NS) + w) * NSEG
            pltpu.sync_copy(zeros, acc_s.at[pl.ds(w * RPT, RPT)])
            plsc.subcore_barrier()

            def seg(s, _, table=table):
                @pl.when(s > 0)
                def _():
                    for u in range(GRP):
                        pltpu.make_async_copy(
                            rows[u], acc_s.at[idx_v.at[0, 0]],
                            ss[u]).wait()

                pltpu.sync_copy(pk.at[gbase + s], idx_v)
                pltpu.sync_copy(vals.at[gbase + s], val_v)

                def group(t, _, table=table):
                    for u in range(GRP):
                        jl = t * GRP + u

                        @pl.when(t > 0)
                        def _(u=u):
                            pltpu.make_async_copy(
                                rows[u], acc_s.at[idx_v.at[0, 0]],
                                ss[u]).wait()

                        pltpu.async_copy(table.at[idx_v.at[jl, 1]],
                                         rows[u], gs[u])
                    for u in range(GRP):
                        jl = t * GRP + u
                        pltpu.make_async_copy(
                            table.at[idx_v.at[0, 1]], rows[u],
                            gs[u]).wait()

                        def scale16(g, _, u=u, jl=jl):
                            vv = val_v[jl, pl.ds(g * 16, 16)]
                            for i in range(16):
                                cc = g * 16 + i
                                v = vv[i]
                                for k in range(D // 16):
                                    sl = pl.ds(k * 16, 16)
                                    rows[u][cc, sl] = rows[u][cc, sl] * v
                            return ()

                        lax.fori_loop(0, C // 16, scale16, ())
                        pltpu.async_copy(rows[u],
                                         acc_s.at[idx_v.at[jl, 0]],
                                         ss[u], add=True)
                    return ()

                lax.fori_loop(0, SEG // GRP, group, ())
                return ()

            lax.fori_loop(0, NSEG, seg, ())
            for u in range(GRP):
                pltpu.make_async_copy(rows[u], acc_s.at[idx_v.at[0, 0]],
                                      ss[u]).wait()
            plsc.subcore_barrier()
            base = (b * L + l) * NPAD
            pltpu.sync_copy(acc_s.at[pl.ds(w * RPT, RPT)],
                            out.at[pl.ds(base + w * RPT, RPT)])
            plsc.subcore_barrier()


@functools.cache
def _get_spmm2():
    mesh = plsc.VectorSubcoreMesh(
        core_axis_name="c", subcore_axis_name="s",
        num_cores=NC, num_subcores=NS)
    return pl.kernel(
        _spmm2_body,
        out_type=[
            jax.ShapeDtypeStruct((B * L * NPAD, D), jnp.float32),
            jax.ShapeDtypeStruct((B * L * NPAD, D), jnp.float32),
        ],
        mesh=mesh,
        scratch_types=[
            pltpu.VMEM((SEG, 2, C), jnp.int32),
            pltpu.VMEM((SEG, C), jnp.float32),
            pltpu.VMEM((C, D), jnp.float32),
            pltpu.VMEM((C, D), jnp.float32),
            pltpu.VMEM((C, D), jnp.float32),
            pltpu.VMEM_SHARED((NPAD, D), jnp.float32),
            pltpu.SemaphoreType.DMA,
            pltpu.SemaphoreType.DMA,
            pltpu.SemaphoreType.DMA,
            pltpu.SemaphoreType.DMA,
            pltpu.SemaphoreType.DMA,
            pltpu.SemaphoreType.DMA,
        ],
    )


BLK = 1000
NBLK = N // BLK
UBLK = USER // BLK


def _stats_body(x0_ref, c1_ref, c2_ref, acc_ref, su_ref, si_ref):
    r = pl.program_id(2)
    a = x0_ref[0] + c1_ref[0, 0] + c2_ref[0, 0]
    acc_ref[0, 0] = a
    p = lax.dot_general(a, a, (((0,), (0,)), ((), ())),
                        precision=lax.Precision.HIGHEST)

    @pl.when(r == 0)
    def _():
        su_ref[...] = jnp.zeros_like(su_ref)
        si_ref[...] = jnp.zeros_like(si_ref)

    is_user = (r < UBLK).astype(jnp.float32)
    su_ref[0, 0] += is_user * p
    si_ref[0, 0] += (1.0 - is_user) * p


_stats = pl.pallas_call(
    _stats_body,
    grid=(L, B, NBLK),
    in_specs=[
        pl.BlockSpec((1, BLK, D), lambda l, b, r: (l, r, 0)),
        pl.BlockSpec((1, 1, BLK, D), lambda l, b, r: (b, l, r, 0)),
        pl.BlockSpec((1, 1, BLK, D), lambda l, b, r: (b, l, r, 0)),
    ],
    out_specs=[
        pl.BlockSpec((1, 1, BLK, D), lambda l, b, r: (l, b, r, 0)),
        pl.BlockSpec((1, 1, D, D), lambda l, b, r: (l, b, 0, 0)),
        pl.BlockSpec((1, 1, D, D), lambda l, b, r: (l, b, 0, 0)),
    ],
    out_shape=[
        jax.ShapeDtypeStruct((L, B, N, D), jnp.float32),
        jax.ShapeDtypeStruct((L, B, D, D), jnp.float32),
        jax.ShapeDtypeStruct((L, B, D, D), jnp.float32),
    ],
)


def _emit_body(acc_ref, hy_ref, s_ref, w_ref, cw_ref, emb_ref, embs_ref):
    emb = jnp.zeros((BLK, D), jnp.float32)
    embs = [jnp.zeros((BLK, D), jnp.float32) for _ in range(B)]
    for l in range(L):
        g = lax.dot_general(hy_ref[l], hy_ref[l], (((0,), (0,)), ((), ())),
                            precision=lax.Precision.HIGHEST)
        ts = []
        for b in range(B):
            m_lb = jnp.dot(g, s_ref[l, b], precision=lax.Precision.HIGHEST)
            t_lb = jnp.dot(acc_ref[l, b], m_lb,
                           precision=lax.Precision.HIGHEST)
            ts.append(t_lb)
            embs[b] = embs[b] + jnp.dot(
                jax.nn.relu(jnp.dot(t_lb, w_ref[l],
                                    precision=lax.Precision.HIGHEST)),
                cw_ref[l], precision=lax.Precision.HIGHEST)
        tm = (ts[0] + ts[1]) * (1.0 / B)
        emb = emb + jnp.dot(
            jax.nn.relu(jnp.dot(tm, w_ref[l],
                                precision=lax.Precision.HIGHEST)),
            cw_ref[l], precision=lax.Precision.HIGHEST)
    emb_ref[...] = emb
    for b in range(B):
        embs_ref[b] = embs[b]


def _make_emit(rows, row_off_blocks):
    nblk = rows // BLK
    return pl.pallas_call(
        _emit_body,
        grid=(nblk,),
        in_specs=[
            pl.BlockSpec((L, B, BLK, D), lambda r: (0, 0, r + row_off_blocks, 0)),
            pl.BlockSpec((L, K, D), lambda r: (0, 0, 0)),
            pl.BlockSpec((L, B, D, D), lambda r: (0, 0, 0, 0)),
            pl.BlockSpec((L, D, D), lambda r: (0, 0, 0)),
            pl.BlockSpec((L, D, D), lambda r: (0, 0, 0)),
        ],
        out_specs=[
            pl.BlockSpec((BLK, D), lambda r: (r, 0)),
            pl.BlockSpec((B, BLK, D), lambda r: (0, r, 0)),
        ],
        out_shape=[
            jax.ShapeDtypeStruct((rows, D), jnp.float32),
            jax.ShapeDtypeStruct((B, rows, D), jnp.float32),
        ],
    )


_emit_user = _make_emit(USER, 0)
_emit_item = _make_emit(ITEM, UBLK)


def kernel(uEmbeds, iEmbeds, uHyper, iHyper, u_w, i_w,
           u_concat_w, i_concat_w, edge_vals, edge_index):
    x0 = jnp.concatenate([uEmbeds, iEmbeds], axis=1)
    xall = x0.reshape(L * N, D)

    pad = EPTP - EPT
    dst = edge_index[:, 0, :].reshape(B, NS, EPT)
    src = edge_index[:, 1, :].reshape(B, NS, EPT)
    zpad = jnp.zeros((B, NS, pad), jnp.int32)
    dst = jnp.concatenate([dst, zpad], axis=2)
    src = jnp.concatenate([src, zpad], axis=2)
    pk = jnp.stack([dst, src], axis=2)
    pk = pk.reshape(B, NS, 2, NSEG, SEG, C).transpose(0, 1, 3, 4, 2, 5)
    pk = pk.reshape(B * NS * NSEG, SEG, 2, C)
    valp = jnp.concatenate(
        [edge_vals.reshape(B, NS, EPT),
         jnp.zeros((B, NS, pad), jnp.float32)], axis=2)
    vals = valp.reshape(B * NS * NSEG, SEG, C)
    zeros = jnp.zeros((RPT, D), jnp.float32)

    cur1, cur2 = _get_spmm2()(xall, pk, vals, zeros)
    c1 = cur1.reshape(B, L, NPAD, D)
    c2 = cur2.reshape(B, L, NPAD, D)

    acc, su, si = _stats(x0, c1, c2)

    ucw = u_concat_w.reshape(L, D, D)
    icw = i_concat_w.reshape(L, D, D)
    ue, ues = _emit_user(acc, uHyper, su, u_w, ucw)
    ie, ies = _emit_item(acc, iHyper, si, i_w, icw)
    return ue, ie, ues, ies
